# Initial kernel scaffold; baseline (speedup 1.0000x reference)
import jax
import jax.numpy as jnp
from jax.experimental import pallas as pl


def kernel(x, edge_index, W1, b1, W2, b2, Wc, bc):
    raise NotImplementedError("write your pallas kernel here")


# speedup vs baseline: 10.4258x; 10.4258x over previous
import dataclasses
import functools

import jax
import jax.numpy as jnp
from jax import lax
from jax.experimental import pallas as pl
from jax.experimental.pallas import tpu as pltpu
from jax.experimental.pallas import tpu_sc as plsc

N = 10000
E = 320000
D = 128
NC, NS = 2, 16
NW = NC * NS
NP = 10240
EP = NW * 10240
CH = 128
RA = 80
SR = 8
SC_ = 1280
ZPT = NP // NS
WRT = 1000

_mesh = plsc.VectorSubcoreMesh(core_axis_name="c", subcore_axis_name="s")
_f32 = jnp.float32


def _sds(shape, dtype=_f32):
    return jax.ShapeDtypeStruct(shape, dtype)


@functools.partial(
    pl.kernel,
    out_type=[_sds((NP,)), _sds((NP,))],
    mesh=_mesh,
    scratch_types=[
        pltpu.VMEM((RA, CH), jnp.int32),
        pltpu.VMEM((CH,), _f32),
        pltpu.VMEM((ZPT,), _f32),
        pltpu.VMEM_SHARED((NP,), _f32),
    ],
)
def _deg_kernel(dsta_hbm, d0_hbm, d1_hbm, idx_v, ones_v, z_v, deg_sh):
    c = lax.axis_index("c")
    s = lax.axis_index("s")
    wid = c * NS + s
    pltpu.sync_copy(dsta_hbm.at[pl.ds(wid * RA, RA)], idx_v)

    @pl.loop(0, CH, step=16)
    def _(i):
        ones_v[pl.ds(i, 16)] = jnp.ones((16,), _f32)

    @pl.loop(0, ZPT, step=16)
    def _(i):
        z_v[pl.ds(i, 16)] = jnp.zeros((16,), _f32)

    pltpu.sync_copy(z_v, deg_sh.at[pl.ds(s * ZPT, ZPT)])
    plsc.subcore_barrier()

    @pl.loop(0, RA)
    def _(j):
        pltpu.sync_copy(ones_v, deg_sh.at[idx_v.at[j]], add=True)

    plsc.subcore_barrier()

    @pl.when((s == 0) & (c == 0))
    def _():
        pltpu.sync_copy(deg_sh, d0_hbm)

    @pl.when((s == 0) & (c == 1))
    def _():
        pltpu.sync_copy(deg_sh, d1_hbm)


@functools.partial(
    pl.kernel,
    out_type=[_sds((N, D)), _sds((N, D))],
    mesh=_mesh,
    scratch_types=[
        pltpu.VMEM((RA, CH), jnp.int32),
        pltpu.VMEM((RA, CH), jnp.int32),
        pltpu.VMEM((CH, D), _f32),
        pltpu.VMEM_SHARED((NP, D), _f32),
    ],
)
def _agg_kernel(p_hbm, srca_hbm, dsta_hbm, a0_hbm, a1_hbm,
                si_v, di_v, rows_v, acc_sh):
    c = lax.axis_index("c")
    s = lax.axis_index("s")
    wid = c * NS + s
    pltpu.sync_copy(srca_hbm.at[pl.ds(wid * RA, RA)], si_v)
    pltpu.sync_copy(dsta_hbm.at[pl.ds(wid * RA, RA)], di_v)

    @pl.loop(0, CH)
    def _(r):
        @pl.loop(0, D, step=16)
        def _(k):
            rows_v[r, pl.ds(k, 16)] = jnp.zeros((16,), _f32)

    for k in range(ZPT // CH):
        pltpu.sync_copy(rows_v, acc_sh.at[pl.ds(s * ZPT + k * CH, CH)])

    plsc.subcore_barrier()

    @pl.loop(0, RA)
    def _(j):
        pltpu.sync_copy(p_hbm.at[si_v.at[j]], rows_v)
        pltpu.sync_copy(rows_v, acc_sh.at[di_v.at[j]], add=True)

    plsc.subcore_barrier()

    @pl.when((s < 10) & (c == 0))
    def _():
        pltpu.sync_copy(acc_sh.at[pl.ds(s * WRT, WRT)],
                        a0_hbm.at[pl.ds(s * WRT, WRT)])

    @pl.when((s < 10) & (c == 1))
    def _():
        pltpu.sync_copy(acc_sh.at[pl.ds(s * WRT, WRT)],
                        a1_hbm.at[pl.ds(s * WRT, WRT)])


_cp = pltpu.CompilerParams()
if "needs_layout_passes" in pltpu.CompilerParams.__dataclass_fields__:
    _cp = dataclasses.replace(_cp, needs_layout_passes=False)


UF = 20096


@functools.partial(
    pl.kernel,
    out_type=_sds((NW * SR, SC_)),
    compiler_params=_cp,
    mesh=_mesh,
    scratch_types=[
        pltpu.VMEM((UF,), _f32),
        pltpu.VMEM((SR, SC_), jnp.int32),
        pltpu.VMEM((SR, SC_), jnp.int32),
        pltpu.VMEM((SR, SC_), _f32),
    ],
)
def _score_kernel(u_hbm, srcs_hbm, dsts_hbm, out_hbm, u_v, s_v, d_v, o_v):
    c = lax.axis_index("c")
    s = lax.axis_index("s")
    wid = c * NS + s
    pltpu.sync_copy(u_hbm, u_v)
    pltpu.sync_copy(srcs_hbm.at[pl.ds(wid * SR, SR)], s_v)
    pltpu.sync_copy(dsts_hbm.at[pl.ds(wid * SR, SR)], d_v)

    @pl.loop(0, SR)
    def _(r):
        @pl.loop(0, SC_, step=16)
        def _(i):
            sv = s_v[r, pl.ds(i, 16)]
            dv = d_v[r, pl.ds(i, 16)]
            a = plsc.load_gather(u_v, [sv * 2])
            b = plsc.load_gather(u_v, [dv * 2 + 1])
            z = a + b
            o_v[r, pl.ds(i, 16)] = 1.0 / (1.0 + jnp.exp(-z))

    pltpu.sync_copy(o_v, out_hbm.at[pl.ds(wid * SR, SR)])


_BR = 2000


def _mm1_body(d0_ref, d1_ref, x_ref, w_ref, p_ref, dinv_ref):
    deg = d0_ref[...] + d1_ref[...] + 1.0
    dinv = lax.rsqrt(deg)
    h = jnp.dot(x_ref[...], w_ref[...], preferred_element_type=_f32)
    p_ref[...] = h * dinv
    dinv_ref[...] = dinv


_mm1 = pl.pallas_call(
    _mm1_body,
    grid=(N // _BR,),
    in_specs=[
        pl.BlockSpec((_BR, 1), lambda i: (i, 0)),
        pl.BlockSpec((_BR, 1), lambda i: (i, 0)),
        pl.BlockSpec((_BR, D), lambda i: (i, 0)),
        pl.BlockSpec((D, D), lambda i: (0, 0)),
    ],
    out_specs=[
        pl.BlockSpec((_BR, D), lambda i: (i, 0)),
        pl.BlockSpec((_BR, 1), lambda i: (i, 0)),
    ],
    out_shape=[_sds((N, D)), _sds((N, 1))],
)


def _mid_body(a0_ref, a1_ref, p_ref, dinv_ref, b_ref, w_ref, out_ref):
    dinv = dinv_ref[...]
    z = dinv * (a0_ref[...] + a1_ref[...] + p_ref[...]) + b_ref[...]
    h = jnp.maximum(z, 0.0)
    out_ref[...] = jnp.dot(h, w_ref[...], preferred_element_type=_f32) * dinv


_mid = pl.pallas_call(
    _mid_body,
    grid=(N // _BR,),
    in_specs=[
        pl.BlockSpec((_BR, D), lambda i: (i, 0)),
        pl.BlockSpec((_BR, D), lambda i: (i, 0)),
        pl.BlockSpec((_BR, D), lambda i: (i, 0)),
        pl.BlockSpec((_BR, 1), lambda i: (i, 0)),
        pl.BlockSpec((1, D), lambda i: (0, 0)),
        pl.BlockSpec((D, D), lambda i: (0, 0)),
    ],
    out_specs=pl.BlockSpec((_BR, D), lambda i: (i, 0)),
    out_shape=_sds((N, D)),
)


def _fin_body(a0_ref, a1_ref, p_ref, dinv_ref, b_ref, wp_ref, br_ref, u_ref):
    dinv = dinv_ref[...]
    z = dinv * (a0_ref[...] + a1_ref[...] + p_ref[...]) + b_ref[...]
    h = jnp.maximum(z, 0.0)
    u_ref[...] = jnp.dot(h, wp_ref[...], preferred_element_type=_f32) + br_ref[...]


_fin = pl.pallas_call(
    _fin_body,
    grid=(N // _BR,),
    in_specs=[
        pl.BlockSpec((_BR, D), lambda i: (i, 0)),
        pl.BlockSpec((_BR, D), lambda i: (i, 0)),
        pl.BlockSpec((_BR, D), lambda i: (i, 0)),
        pl.BlockSpec((_BR, 1), lambda i: (i, 0)),
        pl.BlockSpec((1, D), lambda i: (0, 0)),
        pl.BlockSpec((D, 2), lambda i: (0, 0)),
        pl.BlockSpec((1, 2), lambda i: (0, 0)),
    ],
    out_specs=pl.BlockSpec((_BR, 2), lambda i: (i, 0)),
    out_shape=_sds((N, 2)),
)


@jax.jit
def kernel(x, edge_index, W1, b1, W2, b2, Wc, bc):
    src = edge_index[0].astype(jnp.int32)
    dst = edge_index[1].astype(jnp.int32)
    srcp = jnp.concatenate([src, jnp.zeros((EP - E,), jnp.int32)])
    dstp = jnp.concatenate([dst, jnp.full((EP - E,), N, jnp.int32)])
    src_a = srcp.reshape(NW * RA, CH)
    dst_a = dstp.reshape(NW * RA, CH)
    src_s = srcp.reshape(NW * SR, SC_)
    dst_s = dstp.reshape(NW * SR, SC_)

    d0, d1 = _deg_kernel(dst_a)
    p1, dinv = _mm1(d0[:, None], d1[:, None], x, W1)
    a0, a1 = _agg_kernel(p1, src_a, dst_a)
    p2 = _mid(a0, a1, p1, dinv, b1.reshape(1, D), W2)
    c0, c1 = _agg_kernel(p2, src_a, dst_a)
    wpair = jnp.stack([Wc[:D, 0], Wc[D:, 0]], axis=1)
    brow = jnp.concatenate([bc, jnp.zeros((1,), _f32)]).reshape(1, 2)
    u = _fin(c0, c1, p2, dinv, b2.reshape(1, D), wpair, brow)
    u_flat = jnp.pad(u.reshape(2 * N), (0, UF - 2 * N))
    logits = _score_kernel(u_flat, src_s, dst_s)
    return logits.reshape(EP)[:E].reshape(E, 1)


# speedup vs baseline: 10.4710x; 1.0043x over previous
import dataclasses
import functools

import jax
import jax.numpy as jnp
from jax import lax
from jax.experimental import pallas as pl
from jax.experimental.pallas import tpu as pltpu
from jax.experimental.pallas import tpu_sc as plsc

N = 10000
E = 320000
D = 128
NC, NS = 2, 16
NW = NC * NS
NP = 10240
EP = NW * 10240
CH = 128
RA = 80
SR = 8
SC_ = 1280
ZPT = NP // NS
WRT = 1000

_mesh = plsc.VectorSubcoreMesh(core_axis_name="c", subcore_axis_name="s")
_f32 = jnp.float32


def _sds(shape, dtype=_f32):
    return jax.ShapeDtypeStruct(shape, dtype)


@functools.partial(
    pl.kernel,
    out_type=[_sds((NP,)), _sds((NP,))],
    mesh=_mesh,
    scratch_types=[
        pltpu.VMEM((RA, CH), jnp.int32),
        pltpu.VMEM((CH,), _f32),
        pltpu.VMEM((ZPT,), _f32),
        pltpu.VMEM_SHARED((NP,), _f32),
    ],
)
def _deg_kernel(dsta_hbm, d0_hbm, d1_hbm, idx_v, ones_v, z_v, deg_sh):
    c = lax.axis_index("c")
    s = lax.axis_index("s")
    wid = c * NS + s
    pltpu.sync_copy(dsta_hbm.at[pl.ds(wid * RA, RA)], idx_v)

    @pl.loop(0, CH, step=16)
    def _(i):
        ones_v[pl.ds(i, 16)] = jnp.ones((16,), _f32)

    @pl.loop(0, ZPT, step=16)
    def _(i):
        z_v[pl.ds(i, 16)] = jnp.zeros((16,), _f32)

    pltpu.sync_copy(z_v, deg_sh.at[pl.ds(s * ZPT, ZPT)])
    plsc.subcore_barrier()

    @pl.loop(0, RA)
    def _(j):
        pltpu.sync_copy(ones_v, deg_sh.at[idx_v.at[j]], add=True)

    plsc.subcore_barrier()

    @pl.when((s == 0) & (c == 0))
    def _():
        pltpu.sync_copy(deg_sh, d0_hbm)

    @pl.when((s == 0) & (c == 1))
    def _():
        pltpu.sync_copy(deg_sh, d1_hbm)


@functools.partial(
    pl.kernel,
    out_type=[_sds((N, D)), _sds((N, D))],
    mesh=_mesh,
    scratch_types=[
        pltpu.VMEM((2 * RA, CH), jnp.int32),
        pltpu.VMEM((CH, D), _f32),
        pltpu.VMEM_SHARED((NP, D), _f32),
    ],
)
def _agg_kernel(p_hbm, ed_hbm, a0_hbm, a1_hbm, ei_v, rows_v, acc_sh):
    c = lax.axis_index("c")
    s = lax.axis_index("s")
    wid = c * NS + s
    pltpu.sync_copy(ed_hbm.at[pl.ds(wid * 2 * RA, 2 * RA)], ei_v)

    @pl.loop(0, CH)
    def _(r):
        @pl.loop(0, D, step=16)
        def _(k):
            rows_v[r, pl.ds(k, 16)] = jnp.zeros((16,), _f32)

    for k in range(ZPT // CH):
        pltpu.sync_copy(rows_v, acc_sh.at[pl.ds(s * ZPT + k * CH, CH)])

    plsc.subcore_barrier()

    @pl.loop(0, RA)
    def _(j):
        pltpu.sync_copy(p_hbm.at[ei_v.at[j]], rows_v)
        pltpu.sync_copy(rows_v, acc_sh.at[ei_v.at[RA + j]], add=True)

    plsc.subcore_barrier()

    @pl.when((s < 10) & (c == 0))
    def _():
        pltpu.sync_copy(acc_sh.at[pl.ds(s * WRT, WRT)],
                        a0_hbm.at[pl.ds(s * WRT, WRT)])

    @pl.when((s < 10) & (c == 1))
    def _():
        pltpu.sync_copy(acc_sh.at[pl.ds(s * WRT, WRT)],
                        a1_hbm.at[pl.ds(s * WRT, WRT)])


_cp = pltpu.CompilerParams()
if "needs_layout_passes" in pltpu.CompilerParams.__dataclass_fields__:
    _cp = dataclasses.replace(_cp, needs_layout_passes=False)


UF = 20096


@functools.partial(
    pl.kernel,
    out_type=_sds((NW * SR, SC_)),
    compiler_params=_cp,
    mesh=_mesh,
    scratch_types=[
        pltpu.VMEM((UF,), _f32),
        pltpu.VMEM((SR, SC_), jnp.int32),
        pltpu.VMEM((SR, SC_), jnp.int32),
        pltpu.VMEM((SR, SC_), _f32),
    ],
)
def _score_kernel(u_hbm, srcs_hbm, dsts_hbm, out_hbm, u_v, s_v, d_v, o_v):
    c = lax.axis_index("c")
    s = lax.axis_index("s")
    wid = c * NS + s
    pltpu.sync_copy(u_hbm, u_v)
    pltpu.sync_copy(srcs_hbm.at[pl.ds(wid * SR, SR)], s_v)
    pltpu.sync_copy(dsts_hbm.at[pl.ds(wid * SR, SR)], d_v)

    @pl.loop(0, SR)
    def _(r):
        @pl.loop(0, SC_, step=16)
        def _(i):
            sv = s_v[r, pl.ds(i, 16)]
            dv = d_v[r, pl.ds(i, 16)]
            a = plsc.load_gather(u_v, [sv * 2])
            b = plsc.load_gather(u_v, [dv * 2 + 1])
            z = a + b
            o_v[r, pl.ds(i, 16)] = 1.0 / (1.0 + jnp.exp(-z))

    pltpu.sync_copy(o_v, out_hbm.at[pl.ds(wid * SR, SR)])


_BR = 2000


def _mm1_body(d0_ref, d1_ref, x_ref, w_ref, p_ref, dinv_ref):
    deg = d0_ref[...] + d1_ref[...] + 1.0
    dinv = lax.rsqrt(deg)
    h = jnp.dot(x_ref[...], w_ref[...], preferred_element_type=_f32)
    p_ref[...] = h * dinv
    dinv_ref[...] = dinv


_mm1 = pl.pallas_call(
    _mm1_body,
    grid=(N // _BR,),
    in_specs=[
        pl.BlockSpec((_BR, 1), lambda i: (i, 0)),
        pl.BlockSpec((_BR, 1), lambda i: (i, 0)),
        pl.BlockSpec((_BR, D), lambda i: (i, 0)),
        pl.BlockSpec((D, D), lambda i: (0, 0)),
    ],
    out_specs=[
        pl.BlockSpec((_BR, D), lambda i: (i, 0)),
        pl.BlockSpec((_BR, 1), lambda i: (i, 0)),
    ],
    out_shape=[_sds((N, D)), _sds((N, 1))],
)


def _mid_body(a0_ref, a1_ref, p_ref, dinv_ref, b_ref, w_ref, out_ref):
    dinv = dinv_ref[...]
    z = dinv * (a0_ref[...] + a1_ref[...] + p_ref[...]) + b_ref[...]
    h = jnp.maximum(z, 0.0)
    out_ref[...] = jnp.dot(h, w_ref[...], preferred_element_type=_f32) * dinv


_mid = pl.pallas_call(
    _mid_body,
    grid=(N // _BR,),
    in_specs=[
        pl.BlockSpec((_BR, D), lambda i: (i, 0)),
        pl.BlockSpec((_BR, D), lambda i: (i, 0)),
        pl.BlockSpec((_BR, D), lambda i: (i, 0)),
        pl.BlockSpec((_BR, 1), lambda i: (i, 0)),
        pl.BlockSpec((1, D), lambda i: (0, 0)),
        pl.BlockSpec((D, D), lambda i: (0, 0)),
    ],
    out_specs=pl.BlockSpec((_BR, D), lambda i: (i, 0)),
    out_shape=_sds((N, D)),
)


def _fin_body(a0_ref, a1_ref, p_ref, dinv_ref, b_ref, wp_ref, br_ref, u_ref):
    dinv = dinv_ref[...]
    z = dinv * (a0_ref[...] + a1_ref[...] + p_ref[...]) + b_ref[...]
    h = jnp.maximum(z, 0.0)
    u_ref[...] = jnp.dot(h, wp_ref[...], preferred_element_type=_f32) + br_ref[...]


_fin = pl.pallas_call(
    _fin_body,
    grid=(N // _BR,),
    in_specs=[
        pl.BlockSpec((_BR, D), lambda i: (i, 0)),
        pl.BlockSpec((_BR, D), lambda i: (i, 0)),
        pl.BlockSpec((_BR, D), lambda i: (i, 0)),
        pl.BlockSpec((_BR, 1), lambda i: (i, 0)),
        pl.BlockSpec((1, D), lambda i: (0, 0)),
        pl.BlockSpec((D, 2), lambda i: (0, 0)),
        pl.BlockSpec((1, 2), lambda i: (0, 0)),
    ],
    out_specs=pl.BlockSpec((_BR, 2), lambda i: (i, 0)),
    out_shape=_sds((N, 2)),
)


@jax.jit
def kernel(x, edge_index, W1, b1, W2, b2, Wc, bc):
    src = edge_index[0].astype(jnp.int32)
    dst = edge_index[1].astype(jnp.int32)
    srcp = jnp.concatenate([src, jnp.zeros((EP - E,), jnp.int32)])
    dstp = jnp.concatenate([dst, jnp.full((EP - E,), N, jnp.int32)])
    src_a = srcp.reshape(NW, RA, CH)
    dst_a = dstp.reshape(NW, RA, CH)
    ed_a = jnp.concatenate([src_a, dst_a], axis=1).reshape(NW * 2 * RA, CH)
    src_s = srcp.reshape(NW * SR, SC_)
    dst_s = dstp.reshape(NW * SR, SC_)

    d0, d1 = _deg_kernel(dstp.reshape(NW * RA, CH))
    p1, dinv = _mm1(d0[:, None], d1[:, None], x, W1)
    a0, a1 = _agg_kernel(p1, ed_a)
    p2 = _mid(a0, a1, p1, dinv, b1.reshape(1, D), W2)
    c0, c1 = _agg_kernel(p2, ed_a)
    wpair = jnp.stack([Wc[:D, 0], Wc[D:, 0]], axis=1)
    brow = jnp.concatenate([bc, jnp.zeros((1,), _f32)]).reshape(1, 2)
    u = _fin(c0, c1, p2, dinv, b2.reshape(1, D), wpair, brow)
    u_flat = jnp.pad(u.reshape(2 * N), (0, UF - 2 * N))
    logits = _score_kernel(u_flat, src_s, dst_s)
    return logits.reshape(EP)[:E].reshape(E, 1)


# speedup vs baseline: 30.0444x; 2.8693x over previous
import dataclasses
import functools

import jax
import jax.numpy as jnp
from jax import lax
from jax.experimental import pallas as pl
from jax.experimental.pallas import tpu as pltpu
from jax.experimental.pallas import tpu_sc as plsc

N = 10000
E = 320000
D = 128
NC, NS = 2, 16
NW = NC * NS
NP = 10240
EP = NW * 10240
CH = 128
RA = 80
SR = 8
SC_ = 1280
ZPT = NP // NS
SEG = 16
WRT = 1000

_mesh = plsc.VectorSubcoreMesh(core_axis_name="c", subcore_axis_name="s")
_f32 = jnp.float32


def _sds(shape, dtype=_f32):
    return jax.ShapeDtypeStruct(shape, dtype)


@functools.partial(
    pl.kernel,
    out_type=[_sds((NP,)), _sds((NP,))],
    mesh=_mesh,
    scratch_types=[
        pltpu.VMEM((RA, CH), jnp.int32),
        pltpu.VMEM((CH,), _f32),
        pltpu.VMEM((ZPT,), _f32),
        pltpu.VMEM_SHARED((NP,), _f32),
    ],
)
def _deg_kernel(dsta_hbm, d0_hbm, d1_hbm, idx_v, ones_v, z_v, deg_sh):
    c = lax.axis_index("c")
    s = lax.axis_index("s")
    wid = c * NS + s
    pltpu.sync_copy(dsta_hbm.at[pl.ds(wid * RA, RA)], idx_v)

    @pl.loop(0, CH, step=16)
    def _(i):
        ones_v[pl.ds(i, 16)] = jnp.ones((16,), _f32)

    @pl.loop(0, ZPT, step=16)
    def _(i):
        z_v[pl.ds(i, 16)] = jnp.zeros((16,), _f32)

    pltpu.sync_copy(z_v, deg_sh.at[pl.ds(s * ZPT, ZPT)])
    plsc.subcore_barrier()

    @pl.loop(0, RA)
    def _(j):
        pltpu.sync_copy(ones_v, deg_sh.at[idx_v.at[j]], add=True)

    plsc.subcore_barrier()

    @pl.when((s == 0) & (c == 0))
    def _():
        pltpu.sync_copy(deg_sh, d0_hbm)

    @pl.when((s == 0) & (c == 1))
    def _():
        pltpu.sync_copy(deg_sh, d1_hbm)


@functools.partial(
    pl.kernel,
    out_type=[_sds((N, D)), _sds((N, D))],
    mesh=_mesh,
    scratch_types=[
        pltpu.VMEM((2 * SEG, CH), jnp.int32),
        pltpu.VMEM((CH, D), _f32),
        pltpu.VMEM((CH, D), _f32),
        pltpu.VMEM_SHARED((NP, D), _f32),
        pltpu.SemaphoreType.DMA,
        pltpu.SemaphoreType.DMA,
    ],
)
def _agg_kernel(p_hbm, ed_hbm, a0_hbm, a1_hbm, ei_v, rows0_v, rows1_v,
                acc_sh, sem0, sem1):
    c = lax.axis_index("c")
    s = lax.axis_index("s")
    wid = c * NS + s

    @pl.loop(0, CH)
    def _(r):
        @pl.loop(0, D, step=16)
        def _(k):
            rows0_v[r, pl.ds(k, 16)] = jnp.zeros((16,), _f32)

    for k in range(ZPT // CH):
        pltpu.sync_copy(rows0_v, acc_sh.at[pl.ds(s * ZPT + k * CH, CH)])

    plsc.subcore_barrier()

    @pl.loop(0, RA // SEG)
    def _(g):
        pltpu.sync_copy(ed_hbm.at[pl.ds(wid * 2 * RA + g * 2 * SEG, 2 * SEG)],
                        ei_v)
        pltpu.make_async_copy(p_hbm.at[ei_v.at[0]], rows0_v, sem0).start()

        @pl.loop(0, SEG // 2)
        def _(tt):
            t0 = 2 * tt
            pltpu.make_async_copy(p_hbm.at[ei_v.at[2 * t0]], rows0_v,
                                  sem0).wait()
            pltpu.make_async_copy(p_hbm.at[ei_v.at[2 * t0 + 2]], rows1_v,
                                  sem1).start()
            pltpu.sync_copy(rows0_v, acc_sh.at[ei_v.at[2 * t0 + 1]], add=True)
            pltpu.make_async_copy(p_hbm.at[ei_v.at[2 * t0 + 2]], rows1_v,
                                  sem1).wait()

            @pl.when(t0 + 2 < SEG)
            def _():
                pltpu.make_async_copy(p_hbm.at[ei_v.at[2 * t0 + 4]], rows0_v,
                                      sem0).start()

            pltpu.sync_copy(rows1_v, acc_sh.at[ei_v.at[2 * t0 + 3]], add=True)

    plsc.subcore_barrier()

    @pl.when((s < 10) & (c == 0))
    def _():
        pltpu.sync_copy(acc_sh.at[pl.ds(s * WRT, WRT)],
                        a0_hbm.at[pl.ds(s * WRT, WRT)])

    @pl.when((s < 10) & (c == 1))
    def _():
        pltpu.sync_copy(acc_sh.at[pl.ds(s * WRT, WRT)],
                        a1_hbm.at[pl.ds(s * WRT, WRT)])


_cp = pltpu.CompilerParams()
if "needs_layout_passes" in pltpu.CompilerParams.__dataclass_fields__:
    _cp = dataclasses.replace(_cp, needs_layout_passes=False)


UF = 2 * NP


@functools.partial(
    pl.kernel,
    out_type=_sds((NW * SR, SC_)),
    compiler_params=_cp,
    mesh=_mesh,
    scratch_types=[
        pltpu.VMEM((UF,), _f32),
        pltpu.VMEM((SR, SC_), jnp.int32),
        pltpu.VMEM((SR, SC_), jnp.int32),
        pltpu.VMEM((SR, SC_), _f32),
    ],
)
def _score_kernel(u_hbm, srcs_hbm, dsts_hbm, out_hbm, u_v, s_v, d_v, o_v):
    c = lax.axis_index("c")
    s = lax.axis_index("s")
    wid = c * NS + s
    pltpu.sync_copy(u_hbm, u_v)
    pltpu.sync_copy(srcs_hbm.at[pl.ds(wid * SR, SR)], s_v)
    pltpu.sync_copy(dsts_hbm.at[pl.ds(wid * SR, SR)], d_v)

    @pl.loop(0, SR)
    def _(r):
        @pl.loop(0, SC_, step=16)
        def _(i):
            sv = s_v[r, pl.ds(i, 16)]
            dv = d_v[r, pl.ds(i, 16)]
            a = plsc.load_gather(u_v, [sv * 2])
            b = plsc.load_gather(u_v, [dv * 2 + 1])
            z = a + b
            o_v[r, pl.ds(i, 16)] = 1.0 / (1.0 + jnp.exp(-z))

    pltpu.sync_copy(o_v, out_hbm.at[pl.ds(wid * SR, SR)])


_BR = 2000


def _mm1_body(d0_ref, d1_ref, x_ref, w_ref, p_ref, dinv_ref):
    deg = d0_ref[...] + d1_ref[...] + 1.0
    dinv = lax.rsqrt(deg)
    h = jnp.dot(x_ref[...], w_ref[...], preferred_element_type=_f32)
    p_ref[...] = h * dinv
    dinv_ref[...] = dinv


_mm1 = pl.pallas_call(
    _mm1_body,
    grid=(N // _BR,),
    in_specs=[
        pl.BlockSpec((_BR, 1), lambda i: (i, 0)),
        pl.BlockSpec((_BR, 1), lambda i: (i, 0)),
        pl.BlockSpec((_BR, D), lambda i: (i, 0)),
        pl.BlockSpec((D, D), lambda i: (0, 0)),
    ],
    out_specs=[
        pl.BlockSpec((_BR, D), lambda i: (i, 0)),
        pl.BlockSpec((_BR, 1), lambda i: (i, 0)),
    ],
    out_shape=[_sds((N, D)), _sds((N, 1))],
)


def _mid_body(a0_ref, a1_ref, p_ref, dinv_ref, b_ref, w_ref, out_ref):
    dinv = dinv_ref[...]
    z = dinv * (a0_ref[...] + a1_ref[...] + p_ref[...]) + b_ref[...]
    h = jnp.maximum(z, 0.0)
    out_ref[...] = jnp.dot(h, w_ref[...], preferred_element_type=_f32) * dinv


_mid = pl.pallas_call(
    _mid_body,
    grid=(N // _BR,),
    in_specs=[
        pl.BlockSpec((_BR, D), lambda i: (i, 0)),
        pl.BlockSpec((_BR, D), lambda i: (i, 0)),
        pl.BlockSpec((_BR, D), lambda i: (i, 0)),
        pl.BlockSpec((_BR, 1), lambda i: (i, 0)),
        pl.BlockSpec((1, D), lambda i: (0, 0)),
        pl.BlockSpec((D, D), lambda i: (0, 0)),
    ],
    out_specs=pl.BlockSpec((_BR, D), lambda i: (i, 0)),
    out_shape=_sds((N, D)),
)


def _fin_body(a0_ref, a1_ref, p_ref, dinv_ref, b_ref, wp_ref, br_ref, u_ref):
    dinv = dinv_ref[...]
    z = dinv * (a0_ref[...] + a1_ref[...] + p_ref[...]) + b_ref[...]
    h = jnp.maximum(z, 0.0)
    u_ref[...] = jnp.dot(h, wp_ref[...], preferred_element_type=_f32) + br_ref[...]


_fin = pl.pallas_call(
    _fin_body,
    grid=(N // _BR,),
    in_specs=[
        pl.BlockSpec((_BR, D), lambda i: (i, 0)),
        pl.BlockSpec((_BR, D), lambda i: (i, 0)),
        pl.BlockSpec((_BR, D), lambda i: (i, 0)),
        pl.BlockSpec((_BR, 1), lambda i: (i, 0)),
        pl.BlockSpec((1, D), lambda i: (0, 0)),
        pl.BlockSpec((D, 2), lambda i: (0, 0)),
        pl.BlockSpec((1, 2), lambda i: (0, 0)),
    ],
    out_specs=pl.BlockSpec((_BR, 2), lambda i: (i, 0)),
    out_shape=_sds((N, 2)),
)


@jax.jit
def kernel(x, edge_index, W1, b1, W2, b2, Wc, bc):
    src = edge_index[0].astype(jnp.int32)
    dst = edge_index[1].astype(jnp.int32)
    pad_i = jnp.arange(EP - E, dtype=jnp.int32)
    srcp = jnp.concatenate([src, (pad_i * 37) % N])
    dstp = jnp.concatenate([dst, N + (pad_i % (NP - N))])
    src_a = srcp.reshape(NW, RA, 1, CH)
    dst_a = dstp.reshape(NW, RA, 1, CH)
    ed_a = jnp.concatenate([src_a, dst_a], axis=2).reshape(NW * 2 * RA, CH)
    src_s = srcp.reshape(NW * SR, SC_)
    dst_s = dstp.reshape(NW * SR, SC_)

    d0, d1 = _deg_kernel(dstp.reshape(NW * RA, CH))
    p1, dinv = _mm1(d0[:, None], d1[:, None], x, W1)
    a0, a1 = _agg_kernel(p1, ed_a)
    p2 = _mid(a0, a1, p1, dinv, b1.reshape(1, D), W2)
    c0, c1 = _agg_kernel(p2, ed_a)
    wpair = jnp.stack([Wc[:D, 0], Wc[D:, 0]], axis=1)
    brow = jnp.concatenate([bc, jnp.zeros((1,), _f32)]).reshape(1, 2)
    u = _fin(c0, c1, p2, dinv, b2.reshape(1, D), wpair, brow)
    u_flat = jnp.pad(u.reshape(2 * N), (0, UF - 2 * N))
    logits = _score_kernel(u_flat, src_s, dst_s)
    return logits.reshape(EP)[:E].reshape(E, 1)


# speedup vs baseline: 30.3635x; 1.0106x over previous
import dataclasses
import functools

import jax
import jax.numpy as jnp
from jax import lax
from jax.experimental import pallas as pl
from jax.experimental.pallas import tpu as pltpu
from jax.experimental.pallas import tpu_sc as plsc

N = 10000
E = 320000
D = 128
NC, NS = 2, 16
NW = NC * NS
NP = 10240
EP = NW * 10240
CH = 128
RA = 80
SR = 8
SC_ = 1280
ZPT = NP // NS
SEG = 32
WRT = 1000

_mesh = plsc.VectorSubcoreMesh(core_axis_name="c", subcore_axis_name="s")
_f32 = jnp.float32


def _sds(shape, dtype=_f32):
    return jax.ShapeDtypeStruct(shape, dtype)


@functools.partial(
    pl.kernel,
    out_type=[_sds((NP,)), _sds((NP,))],
    mesh=_mesh,
    scratch_types=[
        pltpu.VMEM((2 * RA, CH), jnp.int32),
        pltpu.VMEM((CH,), _f32),
        pltpu.VMEM((ZPT,), _f32),
        pltpu.VMEM_SHARED((NP,), _f32),
    ],
)
def _deg_kernel(ed_hbm, d0_hbm, d1_hbm, idx_v, ones_v, z_v, deg_sh):
    c = lax.axis_index("c")
    s = lax.axis_index("s")
    wid = c * NS + s
    pltpu.sync_copy(ed_hbm.at[pl.ds(wid * 2 * RA, 2 * RA)], idx_v)

    @pl.loop(0, CH, step=16)
    def _(i):
        ones_v[pl.ds(i, 16)] = jnp.ones((16,), _f32)

    @pl.loop(0, ZPT, step=16)
    def _(i):
        z_v[pl.ds(i, 16)] = jnp.zeros((16,), _f32)

    pltpu.sync_copy(z_v, deg_sh.at[pl.ds(s * ZPT, ZPT)])
    plsc.subcore_barrier()

    @pl.loop(0, RA)
    def _(j):
        pltpu.sync_copy(ones_v, deg_sh.at[idx_v.at[2 * j + 1]], add=True)

    plsc.subcore_barrier()

    @pl.when((s == 0) & (c == 0))
    def _():
        pltpu.sync_copy(deg_sh, d0_hbm)

    @pl.when((s == 0) & (c == 1))
    def _():
        pltpu.sync_copy(deg_sh, d1_hbm)


@functools.partial(
    pl.kernel,
    out_type=[_sds((N, D)), _sds((N, D))],
    mesh=_mesh,
    scratch_types=[
        pltpu.VMEM((2 * SEG, CH), jnp.int32),
        pltpu.VMEM((CH, D), _f32),
        pltpu.VMEM((CH, D), _f32),
        pltpu.VMEM_SHARED((NP, D), _f32),
        pltpu.SemaphoreType.DMA,
        pltpu.SemaphoreType.DMA,
    ],
)
def _agg_kernel(p_hbm, ed_hbm, a0_hbm, a1_hbm, ei_v, rows0_v, rows1_v,
                acc_sh, sem0, sem1):
    c = lax.axis_index("c")
    s = lax.axis_index("s")
    wid = c * NS + s

    @pl.loop(0, CH)
    def _(r):
        @pl.loop(0, D, step=16)
        def _(k):
            rows0_v[r, pl.ds(k, 16)] = jnp.zeros((16,), _f32)

    for k in range(ZPT // CH):
        pltpu.sync_copy(rows0_v, acc_sh.at[pl.ds(s * ZPT + k * CH, CH)])

    plsc.subcore_barrier()

    @pl.loop(0, RA // SEG)
    def _(g):
        pltpu.sync_copy(ed_hbm.at[pl.ds(wid * 2 * RA + g * 2 * SEG, 2 * SEG)],
                        ei_v)
        pltpu.make_async_copy(p_hbm.at[ei_v.at[0]], rows0_v, sem0).start()

        @pl.loop(0, SEG // 2)
        def _(tt):
            t0 = 2 * tt
            pltpu.make_async_copy(p_hbm.at[ei_v.at[2 * t0]], rows0_v,
                                  sem0).wait()
            pltpu.make_async_copy(p_hbm.at[ei_v.at[2 * t0 + 2]], rows1_v,
                                  sem1).start()
            pltpu.sync_copy(rows0_v, acc_sh.at[ei_v.at[2 * t0 + 1]], add=True)
            pltpu.make_async_copy(p_hbm.at[ei_v.at[2 * t0 + 2]], rows1_v,
                                  sem1).wait()

            @pl.when(t0 + 2 < SEG)
            def _():
                pltpu.make_async_copy(p_hbm.at[ei_v.at[2 * t0 + 4]], rows0_v,
                                      sem0).start()

            pltpu.sync_copy(rows1_v, acc_sh.at[ei_v.at[2 * t0 + 3]], add=True)

    plsc.subcore_barrier()

    @pl.when((s < 10) & (c == 0))
    def _():
        pltpu.sync_copy(acc_sh.at[pl.ds(s * WRT, WRT)],
                        a0_hbm.at[pl.ds(s * WRT, WRT)])

    @pl.when((s < 10) & (c == 1))
    def _():
        pltpu.sync_copy(acc_sh.at[pl.ds(s * WRT, WRT)],
                        a1_hbm.at[pl.ds(s * WRT, WRT)])


_cp = pltpu.CompilerParams()
if "needs_layout_passes" in pltpu.CompilerParams.__dataclass_fields__:
    _cp = dataclasses.replace(_cp, needs_layout_passes=False)


UF = 2 * NP


@functools.partial(
    pl.kernel,
    out_type=_sds((NW * 2 * RA, CH)),
    compiler_params=_cp,
    mesh=_mesh,
    scratch_types=[
        pltpu.VMEM((UF,), _f32),
        pltpu.VMEM((2 * RA, CH), jnp.int32),
        pltpu.VMEM((2 * RA, CH), _f32),
    ],
)
def _score_kernel(u_hbm, ed_hbm, out_hbm, u_v, ei_v, o_v):
    c = lax.axis_index("c")
    s = lax.axis_index("s")
    wid = c * NS + s
    pltpu.sync_copy(u_hbm, u_v)
    pltpu.sync_copy(ed_hbm.at[pl.ds(wid * 2 * RA, 2 * RA)], ei_v)

    @pl.loop(0, RA)
    def _(r):
        @pl.loop(0, CH, step=16)
        def _(i):
            sv = ei_v[2 * r, pl.ds(i, 16)]
            dv = ei_v[2 * r + 1, pl.ds(i, 16)]
            a = plsc.load_gather(u_v, [sv * 2])
            b = plsc.load_gather(u_v, [dv * 2 + 1])
            z = a + b
            o_v[2 * r, pl.ds(i, 16)] = 1.0 / (1.0 + jnp.exp(-z))

    pltpu.sync_copy(o_v, out_hbm.at[pl.ds(wid * 2 * RA, 2 * RA)])


_BR = 2000


def _mm1_body(d0_ref, d1_ref, x_ref, w_ref, p_ref, dinv_ref):
    deg = d0_ref[...] + d1_ref[...] + 1.0
    dinv = lax.rsqrt(deg)
    h = jnp.dot(x_ref[...], w_ref[...], preferred_element_type=_f32)
    p_ref[...] = h * dinv
    dinv_ref[...] = dinv


_mm1 = pl.pallas_call(
    _mm1_body,
    grid=(N // _BR,),
    in_specs=[
        pl.BlockSpec((_BR, 1), lambda i: (i, 0)),
        pl.BlockSpec((_BR, 1), lambda i: (i, 0)),
        pl.BlockSpec((_BR, D), lambda i: (i, 0)),
        pl.BlockSpec((D, D), lambda i: (0, 0)),
    ],
    out_specs=[
        pl.BlockSpec((_BR, D), lambda i: (i, 0)),
        pl.BlockSpec((_BR, 1), lambda i: (i, 0)),
    ],
    out_shape=[_sds((N, D)), _sds((N, 1))],
)


def _mid_body(a0_ref, a1_ref, p_ref, dinv_ref, b_ref, w_ref, out_ref):
    dinv = dinv_ref[...]
    z = dinv * (a0_ref[...] + a1_ref[...] + p_ref[...]) + b_ref[...]
    h = jnp.maximum(z, 0.0)
    out_ref[...] = jnp.dot(h, w_ref[...], preferred_element_type=_f32) * dinv


_mid = pl.pallas_call(
    _mid_body,
    grid=(N // _BR,),
    in_specs=[
        pl.BlockSpec((_BR, D), lambda i: (i, 0)),
        pl.BlockSpec((_BR, D), lambda i: (i, 0)),
        pl.BlockSpec((_BR, D), lambda i: (i, 0)),
        pl.BlockSpec((_BR, 1), lambda i: (i, 0)),
        pl.BlockSpec((1, D), lambda i: (0, 0)),
        pl.BlockSpec((D, D), lambda i: (0, 0)),
    ],
    out_specs=pl.BlockSpec((_BR, D), lambda i: (i, 0)),
    out_shape=_sds((N, D)),
)


def _fin_body(a0_ref, a1_ref, p_ref, dinv_ref, b_ref, wp_ref, br_ref, u_ref):
    dinv = dinv_ref[...]
    z = dinv * (a0_ref[...] + a1_ref[...] + p_ref[...]) + b_ref[...]
    h = jnp.maximum(z, 0.0)
    u_ref[...] = jnp.dot(h, wp_ref[...], preferred_element_type=_f32) + br_ref[...]


_fin = pl.pallas_call(
    _fin_body,
    grid=(N // _BR,),
    in_specs=[
        pl.BlockSpec((_BR, D), lambda i: (i, 0)),
        pl.BlockSpec((_BR, D), lambda i: (i, 0)),
        pl.BlockSpec((_BR, D), lambda i: (i, 0)),
        pl.BlockSpec((_BR, 1), lambda i: (i, 0)),
        pl.BlockSpec((1, D), lambda i: (0, 0)),
        pl.BlockSpec((D, 2), lambda i: (0, 0)),
        pl.BlockSpec((1, 2), lambda i: (0, 0)),
    ],
    out_specs=pl.BlockSpec((_BR, 2), lambda i: (i, 0)),
    out_shape=_sds((N, 2)),
)


@jax.jit
def kernel(x, edge_index, W1, b1, W2, b2, Wc, bc):
    src = edge_index[0].astype(jnp.int32)
    dst = edge_index[1].astype(jnp.int32)
    pad_i = jnp.arange(EP - E, dtype=jnp.int32)
    srcp = jnp.concatenate([src, (pad_i * 37) % N])
    dstp = jnp.concatenate([dst, N + (pad_i % (NP - N))])
    src_a = srcp.reshape(NW, RA, 1, CH)
    dst_a = dstp.reshape(NW, RA, 1, CH)
    ed_a = jnp.concatenate([src_a, dst_a], axis=2).reshape(NW * 2 * RA, CH)

    d0, d1 = _deg_kernel(ed_a)
    p1, dinv = _mm1(d0[:, None], d1[:, None], x, W1)
    a0, a1 = _agg_kernel(p1, ed_a)
    p2 = _mid(a0, a1, p1, dinv, b1.reshape(1, D), W2)
    c0, c1 = _agg_kernel(p2, ed_a)
    wpair = jnp.stack([Wc[:D, 0], Wc[D:, 0]], axis=1)
    brow = jnp.concatenate([bc, jnp.zeros((1,), _f32)]).reshape(1, 2)
    u = _fin(c0, c1, p2, dinv, b2.reshape(1, D), wpair, brow)
    u_flat = jnp.pad(u.reshape(2 * N), (0, UF - 2 * N))
    scored = _score_kernel(u_flat, ed_a)
    logits = scored.reshape(NW * RA, 2, CH)[:, 0, :]
    return logits.reshape(EP)[:E].reshape(E, 1)
